# baseline (speedup 1.0000x reference)
import jax
import jax.numpy as jnp
from jax.experimental import pallas as pl

_BLOCK = 8192


def _router_block(x_ref, w1_ref, b1_ref, w2_ref, b2_ref, rb_ref,
                  sel_ref, probs_ref):
    xb = x_ref[...]
    h = jax.lax.dot_general(
        xb, w1_ref[...],
        dimension_numbers=(((1,), (1,)), ((), ())),
        preferred_element_type=jnp.float32)
    h = jnp.maximum(h + b1_ref[...].reshape(1, -1), 0.0)
    lt = jax.lax.dot_general(
        w2_ref[...], h,
        dimension_numbers=(((1,), (1,)), ((), ())),
        preferred_element_type=jnp.float32)
    r = lt.shape[0]
    lt = lt + (b2_ref[...] + rb_ref[...]).reshape(r, 1)
    m = jnp.max(lt, axis=0, keepdims=True)
    e = jnp.exp(lt - m)
    pt = e / jnp.sum(e, axis=0, keepdims=True)
    probs_ref[...] = pt
    mp = jnp.max(pt, axis=0, keepdims=True)
    ids = jax.lax.broadcasted_iota(jnp.int32, pt.shape, 0)
    sel_ref[...] = jnp.min(jnp.where(pt == mp, ids, r), axis=0)


def kernel(x, W1, b1, W2, b2, route_bias):
    n, d = x.shape
    hdim = W1.shape[0]
    r = W2.shape[0]
    blk = _BLOCK if n % _BLOCK == 0 else n
    grid = (n // blk,)
    sel, probs_t = pl.pallas_call(
        _router_block,
        grid=grid,
        in_specs=[
            pl.BlockSpec((blk, d), lambda i: (i, 0)),
            pl.BlockSpec((hdim, d), lambda i: (0, 0)),
            pl.BlockSpec((hdim,), lambda i: (0,)),
            pl.BlockSpec((r, hdim), lambda i: (0, 0)),
            pl.BlockSpec((r,), lambda i: (0,)),
            pl.BlockSpec((r,), lambda i: (0,)),
        ],
        out_specs=[
            pl.BlockSpec((blk,), lambda i: (i,)),
            pl.BlockSpec((r, blk), lambda i: (0, i)),
        ],
        out_shape=[
            jax.ShapeDtypeStruct((n,), jnp.int32),
            jax.ShapeDtypeStruct((r, n), jnp.float32),
        ],
    )(x, W1, b1, W2, b2, route_bias)
    return (sel, probs_t.T)


# speedup vs baseline: 1.0847x; 1.0847x over previous
import jax
import jax.numpy as jnp
from jax.experimental import pallas as pl

_BLOCK = 4096


def _router_block(x_ref, w1_ref, b1_ref, w2_ref, b2_ref, rb_ref,
                  sel_ref, probs_ref):
    xb = x_ref[...]
    h = jax.lax.dot_general(
        xb, w1_ref[...],
        dimension_numbers=(((1,), (1,)), ((), ())),
        preferred_element_type=jnp.float32)
    h = jnp.maximum(h + b1_ref[...].reshape(1, -1), 0.0)
    lt = jax.lax.dot_general(
        w2_ref[...], h,
        dimension_numbers=(((1,), (1,)), ((), ())),
        preferred_element_type=jnp.float32)
    r = lt.shape[0]
    lt = lt + (b2_ref[...] + rb_ref[...]).reshape(r, 1)
    m = jnp.max(lt, axis=0, keepdims=True)
    e = jnp.exp(lt - m)
    pt = e / jnp.sum(e, axis=0, keepdims=True)
    probs_ref[...] = pt
    mp = jnp.max(pt, axis=0, keepdims=True)
    ids = jax.lax.broadcasted_iota(jnp.int32, pt.shape, 0)
    sel_ref[...] = jnp.min(jnp.where(pt == mp, ids, r), axis=0)


def kernel(x, W1, b1, W2, b2, route_bias):
    n, d = x.shape
    hdim = W1.shape[0]
    r = W2.shape[0]
    blk = _BLOCK if n % _BLOCK == 0 else n
    grid = (n // blk,)
    sel, probs_t = pl.pallas_call(
        _router_block,
        grid=grid,
        in_specs=[
            pl.BlockSpec((blk, d), lambda i: (i, 0)),
            pl.BlockSpec((hdim, d), lambda i: (0, 0)),
            pl.BlockSpec((hdim,), lambda i: (0,)),
            pl.BlockSpec((r, hdim), lambda i: (0, 0)),
            pl.BlockSpec((r,), lambda i: (0,)),
            pl.BlockSpec((r,), lambda i: (0,)),
        ],
        out_specs=[
            pl.BlockSpec((blk,), lambda i: (i,)),
            pl.BlockSpec((r, blk), lambda i: (0, i)),
        ],
        out_shape=[
            jax.ShapeDtypeStruct((n,), jnp.int32),
            jax.ShapeDtypeStruct((r, n), jnp.float32),
        ],
    )(x, W1, b1, W2, b2, route_bias)
    return (sel, probs_t.T)
